# baseline (speedup 1.0000x reference)
import functools

import jax
import jax.numpy as jnp
from jax import lax
from jax.experimental import pallas as pl
from jax.experimental.pallas import tpu as pltpu
from jax.experimental.pallas import tpu_sc as plsc

DIM = 768
NUM_EXPERTS = 8
EXPERT_DIM = 1536
TOKENS = 4096
TOP_K = 2
PAIRS = TOKENS * TOP_K
MB = 512
NB = PAIRS // MB
NBP = NB + NUM_EXPERTS
PADROWS = NBP * MB

NCORES = 2
NSUB = 16
NW = NCORES * NSUB
TPW = TOKENS // NW
SUB = 64

def _sc_dispatch(x, pos, w0, w1):
    mesh = plsc.VectorSubcoreMesh(core_axis_name="c", subcore_axis_name="s")

    @functools.partial(
        pl.kernel,
        out_type=(jax.ShapeDtypeStruct((PADROWS, DIM), jnp.float32),
                  jax.ShapeDtypeStruct((PADROWS, 128), jnp.float32)),
        mesh=mesh,
        scratch_types=[
            pltpu.VMEM((SUB,), jnp.int32),
            pltpu.VMEM((SUB,), jnp.int32),
            pltpu.VMEM((SUB, DIM), jnp.float32),
            pltpu.VMEM((SUB,), jnp.float32),
            pltpu.VMEM((SUB,), jnp.float32),
            pltpu.VMEM((SUB, 128), jnp.float32),
            pltpu.VMEM((SUB, 128), jnp.float32),
            pltpu.SemaphoreType.DMA,
        ],
    )
    def body(x_hbm, pos_hbm, w0_hbm, w1_hbm, xs_hbm, ws_hbm,
             idx0_v, idx1_v, rows_v, wv0, wv1, wrow0, wrow1, sem):
        wid = lax.axis_index("s") * NCORES + lax.axis_index("c")
        base = wid * TPW
        for j in range(TPW // SUB):
            b = base + j * SUB
            pltpu.sync_copy(pos_hbm.at[pl.ds(b, SUB)], idx0_v)
            pltpu.sync_copy(pos_hbm.at[pl.ds(TOKENS + b, SUB)], idx1_v)
            pltpu.sync_copy(x_hbm.at[pl.ds(b, SUB)], rows_v)
            pltpu.sync_copy(w0_hbm.at[pl.ds(b, SUB)], wv0)
            pltpu.sync_copy(w1_hbm.at[pl.ds(b, SUB)], wv1)

            def fill(g, carry):
                g16 = g * 16
                a0 = wv0[pl.ds(g16, 16)]
                a1 = wv1[pl.ds(g16, 16)]
                for k in range(16):
                    for v in range(8):
                        sl = pl.ds(v * 16, 16)
                        wrow0[g16 + k, sl] = jnp.broadcast_to(a0[k], (16,))
                        wrow1[g16 + k, sl] = jnp.broadcast_to(a1[k], (16,))
                return carry

            lax.fori_loop(0, SUB // 16, fill, 0)
            c0 = pltpu.async_copy(rows_v, xs_hbm.at[idx0_v], sem)
            c1 = pltpu.async_copy(rows_v, xs_hbm.at[idx1_v], sem)
            c2 = pltpu.async_copy(wrow0, ws_hbm.at[idx0_v], sem)
            c3 = pltpu.async_copy(wrow1, ws_hbm.at[idx1_v], sem)
            c0.wait()
            c1.wait()
            c2.wait()
            c3.wait()

    return body(x, pos, w0, w1)


CSUB = 32


def _sc_combine(ys, pos):
    mesh = plsc.VectorSubcoreMesh(core_axis_name="c", subcore_axis_name="s")
    nchunks = TPW // CSUB

    @functools.partial(
        pl.kernel,
        out_type=jax.ShapeDtypeStruct((TOKENS, DIM), jnp.float32),
        mesh=mesh,
        scratch_types=[
            pltpu.VMEM((CSUB,), jnp.int32),
            pltpu.VMEM((CSUB,), jnp.int32),
            pltpu.VMEM((CSUB,), jnp.int32),
            pltpu.VMEM((CSUB,), jnp.int32),
            pltpu.VMEM((CSUB, DIM), jnp.float32),
            pltpu.VMEM((CSUB, DIM), jnp.float32),
            pltpu.VMEM((CSUB, DIM), jnp.float32),
            pltpu.VMEM((CSUB, DIM), jnp.float32),
            pltpu.SemaphoreType.DMA,
            pltpu.SemaphoreType.DMA,
            pltpu.SemaphoreType.DMA,
        ],
    )
    def body(ys_hbm, pos_hbm, out_hbm,
             idx0a, idx1a, idx0b, idx1b, r0a, r1a, r0b, r1b,
             sga, sgb, sw):
        wid = lax.axis_index("s") * NCORES + lax.axis_index("c")
        base = wid * TPW
        bufs = [(idx0a, idx1a, r0a, r1a, sga), (idx0b, idx1b, r0b, r1b, sgb)]

        def start(j):
            i0, i1, r0, r1, sg = bufs[j % 2]
            b = base + j * CSUB
            pltpu.sync_copy(pos_hbm.at[pl.ds(b, CSUB)], i0)
            pltpu.sync_copy(pos_hbm.at[pl.ds(TOKENS + b, CSUB)], i1)
            return (pltpu.async_copy(ys_hbm.at[i0], r0, sg),
                    pltpu.async_copy(ys_hbm.at[i1], r1, sg))

        pend = start(0)
        wpend = None
        for j in range(nchunks):
            i0, i1, r0, r1, sg = bufs[j % 2]
            if wpend is not None:
                wpend.wait()
                wpend = None
            nxt = start(j + 1) if j + 1 < nchunks else None
            pend[0].wait()
            pend[1].wait()

            def inner(t, c2, r0=r0, r1=r1):
                for v in range(DIM // 16):
                    sl = pl.ds(v * 16, 16)
                    plsc.addupdate(r0.at[t, sl], r1[t, sl])
                return c2

            lax.fori_loop(0, CSUB, inner, 0)
            b = base + j * CSUB
            if j + 1 < nchunks:
                wpend = pltpu.async_copy(r0, out_hbm.at[pl.ds(b, CSUB)], sw)
            else:
                pltpu.sync_copy(r0, out_hbm.at[pl.ds(b, CSUB)])
            pend = nxt

    return body(ys, pos)


def _gmm_kernel(tb_ref, te_ref, tv_ref,
                xs_ref, w1_ref, b1_ref, w2_ref, b2_ref, ws_ref, ys_ref):
    t = pl.program_id(0)

    @pl.when(tv_ref[t] == 1)
    def _go():
        xb = xs_ref[...].astype(jnp.bfloat16)
        w1 = w1_ref[0].astype(jnp.bfloat16)
        w2 = w2_ref[0].astype(jnp.bfloat16)
        h = lax.dot_general(xb, w1, (((1,), (0,)), ((), ())),
                            preferred_element_type=jnp.float32) + b1_ref[0, 0][None, :]
        h = (h * 0.5 * (1.0 + lax.erf(h * 0.7071067811865476))).astype(jnp.bfloat16)
        y = lax.dot_general(h, w2, (((1,), (0,)), ((), ())),
                            preferred_element_type=jnp.float32) + b2_ref[0, 0][None, :]
        ys_ref[...] = y * ws_ref[:, 0:1]


def _grouped_mlp(tb, te, tv, xs, W1, b1, W2, b2, ws):
    grid_spec = pltpu.PrefetchScalarGridSpec(
        num_scalar_prefetch=3,
        grid=(NBP,),
        in_specs=[
            pl.BlockSpec((MB, DIM), lambda t, tb, te, tv: (tb[t], 0)),
            pl.BlockSpec((1, DIM, EXPERT_DIM),
                         lambda t, tb, te, tv: (te[t], 0, 0)),
            pl.BlockSpec((1, 1, EXPERT_DIM),
                         lambda t, tb, te, tv: (te[t], 0, 0)),
            pl.BlockSpec((1, EXPERT_DIM, DIM),
                         lambda t, tb, te, tv: (te[t], 0, 0)),
            pl.BlockSpec((1, 1, DIM),
                         lambda t, tb, te, tv: (te[t], 0, 0)),
            pl.BlockSpec((MB, 128), lambda t, tb, te, tv: (tb[t], 0)),
        ],
        out_specs=pl.BlockSpec((MB, DIM), lambda t, tb, te, tv: (tb[t], 0)),
    )
    return pl.pallas_call(
        _gmm_kernel,
        grid_spec=grid_spec,
        out_shape=jax.ShapeDtypeStruct((PADROWS, DIM), jnp.float32),
    )(tb, te, tv, xs, W1, b1[:, None, :], W2, b2[:, None, :], ws)


def kernel(x, routing_tensor, W1, b1, W2, b2):
    vals, eidx = lax.top_k(routing_tensor, TOP_K)
    e_flat = eidx.astype(jnp.int32).T.reshape(PAIRS)
    onehot = (e_flat[:, None] == jnp.arange(NUM_EXPERTS, dtype=jnp.int32)[None, :]
              ).astype(jnp.int32)
    counts = jnp.sum(onehot, axis=0)
    csum = jnp.cumsum(onehot, axis=0)
    rank = jnp.sum(onehot * (csum - 1), axis=1)

    bcnt = (counts + MB - 1) // MB
    cumb = jnp.cumsum(bcnt).astype(jnp.int32)
    boff = cumb - bcnt
    offp = (boff * MB).astype(jnp.int32)
    pos = (rank + jnp.sum(onehot * offp[None, :], axis=1)).astype(jnp.int32)

    ntp = cumb[-1]
    slot = jnp.arange(NBP, dtype=jnp.int32)
    teq = jnp.sum((slot[:, None] >= cumb[None, :]).astype(jnp.int32), axis=1)
    eidx8 = jnp.arange(NUM_EXPERTS, dtype=jnp.int32)
    emax = jnp.max(jnp.where(counts > 0, eidx8, -1))
    tv = (slot < ntp).astype(jnp.int32)
    tb = jnp.where(slot < ntp, slot, ntp - 1)
    te = jnp.where(slot < ntp, teq, emax)

    xs, ws_plane = _sc_dispatch(x, pos, vals[:, 0], vals[:, 1])
    ys = _grouped_mlp(tb, te, tv, xs, W1, b1, W2, b2, ws_plane)
    return _sc_combine(ys, pos)


# speedup vs baseline: 1.0101x; 1.0101x over previous
import functools

import jax
import jax.numpy as jnp
from jax import lax
from jax.experimental import pallas as pl
from jax.experimental.pallas import tpu as pltpu
from jax.experimental.pallas import tpu_sc as plsc

DIM = 768
NUM_EXPERTS = 8
EXPERT_DIM = 1536
TOKENS = 4096
TOP_K = 2
PAIRS = TOKENS * TOP_K
MB = 352
NBP = -(-PAIRS // MB) + NUM_EXPERTS
PADROWS = NBP * MB

NCORES = 2
NSUB = 16
NW = NCORES * NSUB
TPW = TOKENS // NW
SUB = 64

def _sc_dispatch(x, pos, w0, w1):
    mesh = plsc.VectorSubcoreMesh(core_axis_name="c", subcore_axis_name="s")

    @functools.partial(
        pl.kernel,
        out_type=(jax.ShapeDtypeStruct((PADROWS, DIM), jnp.float32),
                  jax.ShapeDtypeStruct((PADROWS, 128), jnp.float32)),
        mesh=mesh,
        scratch_types=[
            pltpu.VMEM((SUB,), jnp.int32),
            pltpu.VMEM((SUB,), jnp.int32),
            pltpu.VMEM((SUB, DIM), jnp.float32),
            pltpu.VMEM((SUB,), jnp.float32),
            pltpu.VMEM((SUB,), jnp.float32),
            pltpu.VMEM((SUB, 128), jnp.float32),
            pltpu.VMEM((SUB, 128), jnp.float32),
            pltpu.SemaphoreType.DMA,
        ],
    )
    def body(x_hbm, pos_hbm, w0_hbm, w1_hbm, xs_hbm, ws_hbm,
             idx0_v, idx1_v, rows_v, wv0, wv1, wrow0, wrow1, sem):
        wid = lax.axis_index("s") * NCORES + lax.axis_index("c")
        base = wid * TPW
        for j in range(TPW // SUB):
            b = base + j * SUB
            pltpu.sync_copy(pos_hbm.at[pl.ds(b, SUB)], idx0_v)
            pltpu.sync_copy(pos_hbm.at[pl.ds(TOKENS + b, SUB)], idx1_v)
            pltpu.sync_copy(x_hbm.at[pl.ds(b, SUB)], rows_v)
            pltpu.sync_copy(w0_hbm.at[pl.ds(b, SUB)], wv0)
            pltpu.sync_copy(w1_hbm.at[pl.ds(b, SUB)], wv1)

            def fill(g, carry):
                g16 = g * 16
                a0 = wv0[pl.ds(g16, 16)]
                a1 = wv1[pl.ds(g16, 16)]
                for k in range(16):
                    for v in range(8):
                        sl = pl.ds(v * 16, 16)
                        wrow0[g16 + k, sl] = jnp.broadcast_to(a0[k], (16,))
                        wrow1[g16 + k, sl] = jnp.broadcast_to(a1[k], (16,))
                return carry

            lax.fori_loop(0, SUB // 16, fill, 0)
            c0 = pltpu.async_copy(rows_v, xs_hbm.at[idx0_v], sem)
            c1 = pltpu.async_copy(rows_v, xs_hbm.at[idx1_v], sem)
            c2 = pltpu.async_copy(wrow0, ws_hbm.at[idx0_v], sem)
            c3 = pltpu.async_copy(wrow1, ws_hbm.at[idx1_v], sem)
            c0.wait()
            c1.wait()
            c2.wait()
            c3.wait()

    return body(x, pos, w0, w1)


CSUB = 32


def _sc_combine(ys, pos):
    mesh = plsc.VectorSubcoreMesh(core_axis_name="c", subcore_axis_name="s")
    nchunks = TPW // CSUB

    @functools.partial(
        pl.kernel,
        out_type=jax.ShapeDtypeStruct((TOKENS, DIM), jnp.float32),
        mesh=mesh,
        scratch_types=[
            pltpu.VMEM((CSUB,), jnp.int32),
            pltpu.VMEM((CSUB,), jnp.int32),
            pltpu.VMEM((CSUB,), jnp.int32),
            pltpu.VMEM((CSUB,), jnp.int32),
            pltpu.VMEM((CSUB, DIM), jnp.float32),
            pltpu.VMEM((CSUB, DIM), jnp.float32),
            pltpu.VMEM((CSUB, DIM), jnp.float32),
            pltpu.VMEM((CSUB, DIM), jnp.float32),
            pltpu.SemaphoreType.DMA,
            pltpu.SemaphoreType.DMA,
            pltpu.SemaphoreType.DMA,
        ],
    )
    def body(ys_hbm, pos_hbm, out_hbm,
             idx0a, idx1a, idx0b, idx1b, r0a, r1a, r0b, r1b,
             sga, sgb, sw):
        wid = lax.axis_index("s") * NCORES + lax.axis_index("c")
        base = wid * TPW
        bufs = [(idx0a, idx1a, r0a, r1a, sga), (idx0b, idx1b, r0b, r1b, sgb)]

        def start(j):
            i0, i1, r0, r1, sg = bufs[j % 2]
            b = base + j * CSUB
            pltpu.sync_copy(pos_hbm.at[pl.ds(b, CSUB)], i0)
            pltpu.sync_copy(pos_hbm.at[pl.ds(TOKENS + b, CSUB)], i1)
            return (pltpu.async_copy(ys_hbm.at[i0], r0, sg),
                    pltpu.async_copy(ys_hbm.at[i1], r1, sg))

        pend = start(0)
        wpend = None
        for j in range(nchunks):
            i0, i1, r0, r1, sg = bufs[j % 2]
            if wpend is not None:
                wpend.wait()
                wpend = None
            nxt = start(j + 1) if j + 1 < nchunks else None
            pend[0].wait()
            pend[1].wait()

            def inner(t, c2, r0=r0, r1=r1):
                for v in range(DIM // 16):
                    sl = pl.ds(v * 16, 16)
                    plsc.addupdate(r0.at[t, sl], r1[t, sl])
                return c2

            lax.fori_loop(0, CSUB, inner, 0)
            b = base + j * CSUB
            if j + 1 < nchunks:
                wpend = pltpu.async_copy(r0, out_hbm.at[pl.ds(b, CSUB)], sw)
            else:
                pltpu.sync_copy(r0, out_hbm.at[pl.ds(b, CSUB)])
            pend = nxt

    return body(ys, pos)


def _gmm_kernel(tb_ref, te_ref, tv_ref,
                xs_ref, w1_ref, b1_ref, w2_ref, b2_ref, ws_ref, ys_ref):
    t = pl.program_id(0)

    @pl.when(tv_ref[t] == 1)
    def _go():
        xb = xs_ref[...].astype(jnp.bfloat16)
        w1 = w1_ref[0].astype(jnp.bfloat16)
        w2 = w2_ref[0].astype(jnp.bfloat16)
        h = lax.dot_general(xb, w1, (((1,), (0,)), ((), ())),
                            preferred_element_type=jnp.float32) + b1_ref[0, 0][None, :]
        h = (h * 0.5 * (1.0 + lax.erf(h * 0.7071067811865476))).astype(jnp.bfloat16)
        y = lax.dot_general(h, w2, (((1,), (0,)), ((), ())),
                            preferred_element_type=jnp.float32) + b2_ref[0, 0][None, :]
        ys_ref[...] = y * ws_ref[:, 0:1]


def _grouped_mlp(tb, te, tv, xs, W1, b1, W2, b2, ws):
    grid_spec = pltpu.PrefetchScalarGridSpec(
        num_scalar_prefetch=3,
        grid=(NBP,),
        in_specs=[
            pl.BlockSpec((MB, DIM), lambda t, tb, te, tv: (tb[t], 0)),
            pl.BlockSpec((1, DIM, EXPERT_DIM),
                         lambda t, tb, te, tv: (te[t], 0, 0)),
            pl.BlockSpec((1, 1, EXPERT_DIM),
                         lambda t, tb, te, tv: (te[t], 0, 0)),
            pl.BlockSpec((1, EXPERT_DIM, DIM),
                         lambda t, tb, te, tv: (te[t], 0, 0)),
            pl.BlockSpec((1, 1, DIM),
                         lambda t, tb, te, tv: (te[t], 0, 0)),
            pl.BlockSpec((MB, 128), lambda t, tb, te, tv: (tb[t], 0)),
        ],
        out_specs=pl.BlockSpec((MB, DIM), lambda t, tb, te, tv: (tb[t], 0)),
    )
    return pl.pallas_call(
        _gmm_kernel,
        grid_spec=grid_spec,
        out_shape=jax.ShapeDtypeStruct((PADROWS, DIM), jnp.float32),
    )(tb, te, tv, xs, W1, b1[:, None, :], W2, b2[:, None, :], ws)


def kernel(x, routing_tensor, W1, b1, W2, b2):
    vals, eidx = lax.top_k(routing_tensor, TOP_K)
    e_flat = eidx.astype(jnp.int32).T.reshape(PAIRS)
    onehot = (e_flat[:, None] == jnp.arange(NUM_EXPERTS, dtype=jnp.int32)[None, :]
              ).astype(jnp.int32)
    counts = jnp.sum(onehot, axis=0)
    csum = jnp.cumsum(onehot, axis=0)
    rank = jnp.sum(onehot * (csum - 1), axis=1)

    bcnt = (counts + MB - 1) // MB
    cumb = jnp.cumsum(bcnt).astype(jnp.int32)
    boff = cumb - bcnt
    offp = (boff * MB).astype(jnp.int32)
    pos = (rank + jnp.sum(onehot * offp[None, :], axis=1)).astype(jnp.int32)

    ntp = cumb[-1]
    slot = jnp.arange(NBP, dtype=jnp.int32)
    teq = jnp.sum((slot[:, None] >= cumb[None, :]).astype(jnp.int32), axis=1)
    eidx8 = jnp.arange(NUM_EXPERTS, dtype=jnp.int32)
    emax = jnp.max(jnp.where(counts > 0, eidx8, -1))
    tv = (slot < ntp).astype(jnp.int32)
    tb = jnp.where(slot < ntp, slot, ntp - 1)
    te = jnp.where(slot < ntp, teq, emax)

    xs, ws_plane = _sc_dispatch(x, pos, vals[:, 0], vals[:, 1])
    ys = _grouped_mlp(tb, te, tv, xs, W1, b1, W2, b2, ws_plane)
    return _sc_combine(ys, pos)
